# baseline (speedup 1.0000x reference)
import jax
import jax.numpy as jnp
from jax.experimental import pallas as pl
from jax.experimental.pallas import tpu as pltpu


def _lookup_body(pe_ref, out_ref, sem):
    pltpu.make_async_copy(pe_ref, out_ref, sem).start()
    pltpu.make_async_copy(pe_ref, out_ref, sem).wait()


def kernel(x, pos_embed):
    seq_len = x.shape[0]
    d_model = pos_embed.shape[1]
    pe3 = pos_embed[:seq_len].reshape(seq_len, 1, d_model)
    return pl.pallas_call(
        _lookup_body,
        in_specs=[pl.BlockSpec(memory_space=pl.ANY)],
        out_specs=pl.BlockSpec(memory_space=pl.ANY),
        out_shape=jax.ShapeDtypeStruct((seq_len, 1, d_model), pos_embed.dtype),
        scratch_shapes=[pltpu.SemaphoreType.DMA],
    )(pe3)
